# baseline (speedup 1.0000x reference)
import functools

import jax
import jax.numpy as jnp
from jax import lax
from jax.experimental import pallas as pl
from jax.experimental.pallas import tpu as pltpu
from jax.experimental.pallas import tpu_sc as plsc

N = 10000
E = 160000
H = 128
DE = 16
NC = 2
NL = 2
NG = 128
NCLS = 10
G = NC + 1

CH = 64
EPC = E // 2
QPC = EPC // CH
NSUB = 16
QPW = (QPC + NSUB - 1) // NSUB
RPS = 624


NBUF = 3
EBUF = 2
QPB = E // CH
EWS = 16777216.0


def _make_msg_call(table_rows, offs):
    mesh = plsc.VectorSubcoreMesh(core_axis_name="c", subcore_axis_name="s")

    @functools.partial(
        pl.kernel,
        mesh=mesh,
        out_type=jax.ShapeDtypeStruct((2, G, N, H), jnp.float32),
        scratch_types=[
            pltpu.VMEM((NBUF * 3 + 1, CH), jnp.int32),
            pltpu.VMEM((NBUF, CH), jnp.int32),
            pltpu.VMEM((NBUF * CH, H), jnp.float32),
            pltpu.VMEM((EBUF * CH, H), jnp.float32),
            pltpu.VMEM_SHARED((N, H), jnp.float32),
            pltpu.SemaphoreType.DMA((NBUF,)),
            pltpu.SemaphoreType.DMA((NBUF,)),
            pltpu.SemaphoreType.DMA((EBUF,)),
            pltpu.SemaphoreType.DMA((NBUF,)),
        ],
    )
    def msg(pk_hbm, eenc_hbm, table_hbm, out_hbm,
            pk_v, dsts_v, rows_v, ea_v, agg_sh, s_pk, s_g, s_ea, s_sc):
        cid = lax.axis_index("c")
        sid = lax.axis_index("s")

        rbase = pl.multiple_of(sid * RPS, 8)

        for g in range(G):
            goff = jnp.int32(offs[g])

            def q_of(t):
                return g * QPB + cid * QPC + t * NSUB + sid

            def valid(t):
                return (t * NSUB + sid) < QPC

            def issue_pk(t):
                b = jnp.int32(t) % NBUF
                pltpu.async_copy(pk_hbm.at[q_of(t)],
                                 pk_v.at[pl.ds(b * 3, 3)], s_pk.at[b])

            def issue_gather(t):
                b = jnp.int32(t) % NBUF
                sbase = pl.multiple_of(b * CH, 8)
                pltpu.make_async_copy(pk_hbm.at[q_of(t)],
                                      pk_v.at[pl.ds(b * 3, 3)],
                                      s_pk.at[b]).wait()
                if offs[g]:
                    for k in range(CH // 16):
                        sl = pl.ds(k * 16, 16)
                        pk_v[b * 3, sl] = pk_v[b * 3, sl] + goff
                pltpu.async_copy(table_hbm.at[pk_v.at[b * 3]],
                                 rows_v.at[pl.ds(sbase, CH)], s_g.at[b])

            def issue_ea(t):
                be = jnp.int32(t) % EBUF
                pltpu.async_copy(
                    eenc_hbm.at[pl.ds(pl.multiple_of(q_of(t) * CH, CH), CH)],
                    ea_v.at[pl.ds(pl.multiple_of(be * CH, 8), CH)],
                    s_ea.at[be])

            def wait_scatter(t):
                b = jnp.int32(t) % NBUF
                sbase = pl.multiple_of(b * CH, 8)
                pltpu.make_async_copy(rows_v.at[pl.ds(sbase, CH)],
                                      agg_sh.at[dsts_v.at[b]],
                                      s_sc.at[b]).wait()

            def compute(t):
                b = jnp.int32(t) % NBUF
                be = jnp.int32(t) % EBUF
                sbase = pl.multiple_of(b * CH, 8)
                ebase = pl.multiple_of(be * CH, 8)
                pltpu.make_async_copy(table_hbm.at[pk_v.at[b * 3]],
                                      rows_v.at[pl.ds(sbase, CH)],
                                      s_g.at[b]).wait()
                pltpu.make_async_copy(
                    eenc_hbm.at[pl.ds(pl.multiple_of(q_of(t) * CH, CH), CH)],
                    ea_v.at[pl.ds(ebase, CH)], s_ea.at[be]).wait()
                for k in range(CH // 16):
                    dsts_v[b, pl.ds(k * 16, 16)] = \
                        pk_v[b * 3 + 1, pl.ds(k * 16, 16)]
                wrow = b * 3 + 2

                def edge(e, _):
                    wv = pk_v[wrow, pl.ds(e, 16)].astype(jnp.float32)
                    wj = jnp.full((16,), wv[0], jnp.float32)
                    r = sbase + e
                    re = ebase + e
                    for k in range(H // 16):
                        sl = pl.ds(k * 16, 16)
                        rows_v[r, sl] = jnp.maximum(
                            rows_v[r, sl] + ea_v[re, sl], 0.0) * wj
                    return ()

                lax.fori_loop(0, CH, edge, ())
                pltpu.async_copy(rows_v.at[pl.ds(sbase, CH)],
                                 agg_sh.at[dsts_v.at[b]], s_sc.at[b],
                                 add=True)

            z16 = jnp.zeros((16,), jnp.float32)

            def zrow(i, _):
                for k in range(H // 16):
                    rows_v[i, pl.ds(k * 16, 16)] = z16
                return ()

            lax.fori_loop(0, CH, zrow, ())
            zsrc = rows_v.at[pl.ds(0, CH)]
            for j in range(RPS // CH):
                pltpu.sync_copy(zsrc, agg_sh.at[pl.ds(rbase + j * CH, CH)])
            if RPS % CH:
                pltpu.sync_copy(rows_v.at[pl.ds(0, RPS % CH)],
                                agg_sh.at[pl.ds(rbase + (RPS // CH) * CH,
                                                RPS % CH)])

            @pl.when(sid == NSUB - 1)
            def _():
                pltpu.sync_copy(rows_v.at[pl.ds(0, 16)],
                                agg_sh.at[pl.ds(NSUB * RPS, 16)])

            plsc.subcore_barrier()

            issue_pk(0)
            issue_pk(1)
            issue_ea(0)
            issue_gather(0)

            def step(t, _):
                @pl.when(valid(t + 2))
                def _():
                    issue_pk(t + 2)

                @pl.when((t >= 2) & valid(t - 2))
                def _():
                    wait_scatter(t - 2)

                @pl.when(valid(t + 1))
                def _():
                    issue_gather(t + 1)
                    issue_ea(t + 1)

                @pl.when(valid(t))
                def _():
                    compute(t)

                return ()

            lax.fori_loop(0, QPW, step, ())

            for d in (QPW - 2, QPW - 1):
                @pl.when(valid(d))
                def _():
                    wait_scatter(d)

            plsc.subcore_barrier()
            pltpu.sync_copy(agg_sh.at[pl.ds(rbase, RPS)],
                            out_hbm.at[cid, g, pl.ds(rbase, RPS)])

            @pl.when(sid == NSUB - 1)
            def _():
                pltpu.sync_copy(agg_sh.at[pl.ds(NSUB * RPS, 16)],
                                out_hbm.at[cid, g, pl.ds(NSUB * RPS, 16)])

            plsc.subcore_barrier()

    return msg


def _eenc_body(ea_ref, We_ref, be_ref, o_ref):
    o_ref[...] = (jnp.dot(ea_ref[...], We_ref[...],
                          preferred_element_type=jnp.float32) + be_ref[...])


def _edge_encode(ea_all, We, be):
    RB = 4000
    grid = (3 * E // RB,)
    return pl.pallas_call(
        _eenc_body,
        grid=grid,
        in_specs=[
            pl.BlockSpec((RB, DE), lambda i: (i, 0)),
            pl.BlockSpec((DE, H), lambda i: (0, 0)),
            pl.BlockSpec((1, H), lambda i: (0, 0)),
        ],
        out_specs=pl.BlockSpec((RB, H), lambda i: (i, 0)),
        out_shape=jax.ShapeDtypeStruct((3 * E, H), jnp.float32),
    )(ea_all, We, be.reshape(1, H))


def _dense_body(RB, hin_ref, a0_ref, a1_ref, sc_ref, W1_ref, b1_ref,
                W2_ref, b2_ref, gm_ref, bt_ref, o_ref):
    h = hin_ref[...].reshape(RB, H)
    a = (a0_ref[...].reshape(RB, H) + a1_ref[...].reshape(RB, H)) * (1.0 / EWS)
    z = sc_ref[...].reshape(1, H) * h + a
    t = jnp.maximum(jnp.dot(z, W1_ref[...].reshape(H, H),
                            preferred_element_type=jnp.float32)
                    + b1_ref[...].reshape(1, H), 0.0)
    t = (jnp.dot(t, W2_ref[...].reshape(H, H),
                 preferred_element_type=jnp.float32)
         + b2_ref[...].reshape(1, H))
    t = jnp.maximum(gm_ref[...].reshape(1, H) * t
                    + bt_ref[...].reshape(1, H), 0.0)
    o_ref[...] = (t + h).reshape(1, RB, H)


def _dense_layer(l, hin, aggp, W1, b1, W2, b2, gamma, beta, eps):
    RB = 2000
    NB = N // RB
    scl = jnp.broadcast_to((1.0 + eps[:, l])[:, None, None], (G, 1, H))

    def v3(p):
        return p[:, l][:, None, :]

    if l == 0:
        hin_spec = pl.BlockSpec((RB, H), lambda g, i: (i, 0))
    else:
        hin_spec = pl.BlockSpec((1, RB, H), lambda g, i: (g, i, 0))

    vspec = pl.BlockSpec((1, 1, H), lambda g, i: (g, 0, 0))
    return pl.pallas_call(
        functools.partial(_dense_body, RB),
        grid=(G, NB),
        in_specs=[
            hin_spec,
            pl.BlockSpec((1, 1, RB, H), lambda g, i: (0, g, i, 0)),
            pl.BlockSpec((1, 1, RB, H), lambda g, i: (1, g, i, 0)),
            vspec,
            pl.BlockSpec((1, H, H), lambda g, i: (g, 0, 0)),
            vspec,
            pl.BlockSpec((1, H, H), lambda g, i: (g, 0, 0)),
            vspec,
            vspec,
            vspec,
        ],
        out_specs=pl.BlockSpec((1, RB, H), lambda g, i: (g, i, 0)),
        out_shape=jax.ShapeDtypeStruct((G, N, H), jnp.float32),
    )(hin, aggp, aggp, scl, W1[:, l], v3(b1), W2[:, l], v3(b2),
      v3(gamma), v3(beta))


def _pool_body(RB, h_ref, b_ref, s_ref, c_ref):
    g = pl.program_id(0)
    i = pl.program_id(1)
    h = h_ref[...].reshape(RB, H)
    b = b_ref[...]
    oh = (b == lax.broadcasted_iota(jnp.int32, (RB, NG), 1)).astype(jnp.float32)
    sblk = jnp.dot(oh.T, h, preferred_element_type=jnp.float32)

    @pl.when(i == 0)
    def _():
        s_ref[...] = jnp.zeros_like(s_ref)

    s_ref[...] += sblk.reshape(1, NG, H)

    @pl.when((g == 0) & (i == 0))
    def _():
        c_ref[...] = jnp.zeros_like(c_ref)

    @pl.when(g == 0)
    def _():
        cs = jnp.sum(oh, axis=0)
        c_ref[...] += jnp.broadcast_to(cs[:, None], (NG, H))


def _pool(h2, batch2d):
    RB = 2000
    NB = N // RB
    return pl.pallas_call(
        functools.partial(_pool_body, RB),
        grid=(G, NB),
        in_specs=[
            pl.BlockSpec((1, RB, H), lambda g, i: (g, i, 0)),
            pl.BlockSpec((RB, 1), lambda g, i: (i, 0)),
        ],
        out_specs=[
            pl.BlockSpec((1, NG, H), lambda g, i: (g, 0, 0)),
            pl.BlockSpec((NG, H), lambda g, i: (0, 0)),
        ],
        out_shape=[
            jax.ShapeDtypeStruct((G, NG, H), jnp.float32),
            jax.ShapeDtypeStruct((NG, H), jnp.float32),
        ],
    )(h2, batch2d)


def _head_body(s_ref, c_ref, Wm_ref, bm_ref, Wf_ref, bf_ref, o_ref):
    c = jnp.maximum(c_ref[...], 1.0)
    acc = jnp.zeros((NG, H), jnp.float32)
    for g in range(G):
        hg = s_ref[g] / c
        hg = jnp.maximum(jnp.dot(hg, Wm_ref[g],
                                 preferred_element_type=jnp.float32)
                         + bm_ref[g], 0.0)
        acc = acc + hg
    acc = acc * (1.0 / G)
    o_ref[...] = (jnp.dot(acc, Wf_ref[...],
                          preferred_element_type=jnp.float32) + bf_ref[...])


def _head(s, c, Wm, bm, Wf, bf):
    return pl.pallas_call(
        _head_body,
        out_shape=jax.ShapeDtypeStruct((NG, NCLS), jnp.float32),
    )(s, c, Wm, bm[:, None, :], Wf, bf.reshape(1, NCLS))


def kernel(x, edge_attr, edge_weight, cand_edge_attr, cand_edge_weight,
           W1, b1, W2, b2, gamma, beta, eps, We, be, Wm, bm, Wf, bf,
           edge_index, cand_edge_index, batch):
    ei_all = jnp.concatenate([edge_index[None], cand_edge_index], axis=0)
    src_all = ei_all[:, 0, :].reshape(3 * E)
    dst_all = ei_all[:, 1, :].reshape(3 * E)
    ew_all = jnp.concatenate([edge_weight[None], cand_edge_weight],
                             axis=0).reshape(3 * E)
    ea_all = jnp.concatenate([edge_attr[None], cand_edge_attr],
                             axis=0).reshape(3 * E, DE)
    pk = jnp.stack([
        src_all.reshape(3 * QPB, CH),
        dst_all.reshape(3 * QPB, CH),
        (ew_all * EWS).astype(jnp.int32).reshape(3 * QPB, CH),
    ], axis=1)

    eenc = _edge_encode(ea_all, We, be)

    msg0 = _make_msg_call(N, (0, 0, 0))
    aggp0 = msg0(pk, eenc, x)
    h1 = _dense_layer(0, x, aggp0, W1, b1, W2, b2, gamma, beta, eps)

    msg1 = _make_msg_call(G * N, (0, N, 2 * N))
    aggp1 = msg1(pk, eenc, h1.reshape(G * N, H))
    h2 = _dense_layer(1, h1, aggp1, W1, b1, W2, b2, gamma, beta, eps)

    s, c = _pool(h2, batch.reshape(N, 1).astype(jnp.int32))
    return _head(s, c, Wm, bm, Wf, bf)


# speedup vs baseline: 1.1545x; 1.1545x over previous
import functools

import jax
import jax.numpy as jnp
from jax import lax
from jax.experimental import pallas as pl
from jax.experimental.pallas import tpu as pltpu
from jax.experimental.pallas import tpu_sc as plsc

N = 10000
E = 160000
H = 128
DE = 16
NC = 2
NL = 2
NG = 128
NCLS = 10
G = NC + 1

CH = 64
EPC = E // 2
QPC = EPC // CH
NSUB = 16
QPW = (QPC + NSUB - 1) // NSUB
RPS = 624


NBUF = 3
EBUF = 2
QPB = E // CH
EWS = 16777216.0


def _make_msg_call(table_rows, offs):
    mesh = plsc.VectorSubcoreMesh(core_axis_name="c", subcore_axis_name="s")

    @functools.partial(
        pl.kernel,
        mesh=mesh,
        out_type=jax.ShapeDtypeStruct((2, G, N, H), jnp.float32),
        scratch_types=[
            pltpu.VMEM((NBUF * 3 + 1, CH), jnp.int32),
            pltpu.VMEM((NBUF, CH), jnp.int32),
            pltpu.VMEM((NBUF * CH, H), jnp.float32),
            pltpu.VMEM((EBUF * CH, H), jnp.float32),
            pltpu.VMEM_SHARED((N, H), jnp.float32),
            pltpu.SemaphoreType.DMA((NBUF,)),
            pltpu.SemaphoreType.DMA((NBUF,)),
            pltpu.SemaphoreType.DMA((EBUF,)),
            pltpu.SemaphoreType.DMA((NBUF,)),
        ],
    )
    def msg(pk_hbm, eenc_hbm, table_hbm, out_hbm,
            pk_v, dsts_v, rows_v, ea_v, agg_sh, s_pk, s_g, s_ea, s_sc):
        cid = lax.axis_index("c")
        sid = lax.axis_index("s")

        rbase = pl.multiple_of(sid * RPS, 8)

        for g in range(G):
            goff = jnp.int32(offs[g])

            def q_of(t):
                return g * QPB + cid * QPC + t * NSUB + sid

            def valid(t):
                return (t * NSUB + sid) < QPC

            def issue_pk(t):
                b = jnp.int32(t) % NBUF
                pltpu.async_copy(pk_hbm.at[q_of(t)],
                                 pk_v.at[pl.ds(b * 3, 3)], s_pk.at[b])

            def issue_gather(t):
                b = jnp.int32(t) % NBUF
                sbase = pl.multiple_of(b * CH, 8)
                pltpu.make_async_copy(pk_hbm.at[q_of(t)],
                                      pk_v.at[pl.ds(b * 3, 3)],
                                      s_pk.at[b]).wait()
                if offs[g]:
                    for k in range(CH // 16):
                        sl = pl.ds(k * 16, 16)
                        pk_v[b * 3, sl] = pk_v[b * 3, sl] + goff
                pltpu.async_copy(table_hbm.at[pk_v.at[b * 3]],
                                 rows_v.at[pl.ds(sbase, CH)], s_g.at[b])

            def issue_ea(t):
                be = jnp.int32(t) % EBUF
                pltpu.async_copy(
                    eenc_hbm.at[pl.ds(pl.multiple_of(q_of(t) * CH, CH), CH)],
                    ea_v.at[pl.ds(pl.multiple_of(be * CH, 8), CH)],
                    s_ea.at[be])

            def wait_scatter(t):
                b = jnp.int32(t) % NBUF
                sbase = pl.multiple_of(b * CH, 8)
                pltpu.make_async_copy(rows_v.at[pl.ds(sbase, CH)],
                                      agg_sh.at[dsts_v.at[b]],
                                      s_sc.at[b]).wait()

            def compute(t):
                b = jnp.int32(t) % NBUF
                be = jnp.int32(t) % EBUF
                sbase = pl.multiple_of(b * CH, 8)
                ebase = pl.multiple_of(be * CH, 8)
                pltpu.make_async_copy(table_hbm.at[pk_v.at[b * 3]],
                                      rows_v.at[pl.ds(sbase, CH)],
                                      s_g.at[b]).wait()
                pltpu.make_async_copy(
                    eenc_hbm.at[pl.ds(pl.multiple_of(q_of(t) * CH, CH), CH)],
                    ea_v.at[pl.ds(ebase, CH)], s_ea.at[be]).wait()
                for k in range(CH // 16):
                    dsts_v[b, pl.ds(k * 16, 16)] = \
                        pk_v[b * 3 + 1, pl.ds(k * 16, 16)]
                wrow = b * 3 + 2

                def egroup(gi, _):
                    e0 = gi * 16
                    wv = pk_v[wrow, pl.ds(e0, 16)].astype(jnp.float32)
                    for j in range(16):
                        wj = jnp.full((16,), wv[j], jnp.float32)
                        r = sbase + e0 + j
                        re = ebase + e0 + j
                        for k in range(H // 16):
                            sl = pl.ds(k * 16, 16)
                            rows_v[r, sl] = jnp.maximum(
                                rows_v[r, sl] + ea_v[re, sl], 0.0) * wj
                    return ()

                lax.fori_loop(0, CH // 16, egroup, ())
                pltpu.async_copy(rows_v.at[pl.ds(sbase, CH)],
                                 agg_sh.at[dsts_v.at[b]], s_sc.at[b],
                                 add=True)

            z16 = jnp.zeros((16,), jnp.float32)

            def zrow(i, _):
                for k in range(H // 16):
                    rows_v[i, pl.ds(k * 16, 16)] = z16
                return ()

            lax.fori_loop(0, CH, zrow, ())
            zsrc = rows_v.at[pl.ds(0, CH)]
            for j in range(RPS // CH):
                pltpu.sync_copy(zsrc, agg_sh.at[pl.ds(rbase + j * CH, CH)])
            if RPS % CH:
                pltpu.sync_copy(rows_v.at[pl.ds(0, RPS % CH)],
                                agg_sh.at[pl.ds(rbase + (RPS // CH) * CH,
                                                RPS % CH)])

            @pl.when(sid == NSUB - 1)
            def _():
                pltpu.sync_copy(rows_v.at[pl.ds(0, 16)],
                                agg_sh.at[pl.ds(NSUB * RPS, 16)])

            plsc.subcore_barrier()

            issue_pk(0)
            issue_pk(1)
            issue_ea(0)
            issue_gather(0)

            def step(t, _):
                @pl.when(valid(t + 2))
                def _():
                    issue_pk(t + 2)

                @pl.when((t >= 2) & valid(t - 2))
                def _():
                    wait_scatter(t - 2)

                @pl.when(valid(t + 1))
                def _():
                    issue_gather(t + 1)
                    issue_ea(t + 1)

                @pl.when(valid(t))
                def _():
                    compute(t)

                return ()

            lax.fori_loop(0, QPW, step, ())

            for d in (QPW - 2, QPW - 1):
                @pl.when(valid(d))
                def _():
                    wait_scatter(d)

            plsc.subcore_barrier()
            pltpu.sync_copy(agg_sh.at[pl.ds(rbase, RPS)],
                            out_hbm.at[cid, g, pl.ds(rbase, RPS)])

            @pl.when(sid == NSUB - 1)
            def _():
                pltpu.sync_copy(agg_sh.at[pl.ds(NSUB * RPS, 16)],
                                out_hbm.at[cid, g, pl.ds(NSUB * RPS, 16)])

            plsc.subcore_barrier()

    return msg


def _eenc_body(ea_ref, We_ref, be_ref, o_ref):
    o_ref[...] = (jnp.dot(ea_ref[...], We_ref[...],
                          preferred_element_type=jnp.float32) + be_ref[...])


def _edge_encode(ea_all, We, be):
    RB = 4000
    grid = (3 * E // RB,)
    return pl.pallas_call(
        _eenc_body,
        grid=grid,
        in_specs=[
            pl.BlockSpec((RB, DE), lambda i: (i, 0)),
            pl.BlockSpec((DE, H), lambda i: (0, 0)),
            pl.BlockSpec((1, H), lambda i: (0, 0)),
        ],
        out_specs=pl.BlockSpec((RB, H), lambda i: (i, 0)),
        out_shape=jax.ShapeDtypeStruct((3 * E, H), jnp.float32),
    )(ea_all, We, be.reshape(1, H))


def _dense_body(RB, hin_ref, a0_ref, a1_ref, sc_ref, W1_ref, b1_ref,
                W2_ref, b2_ref, gm_ref, bt_ref, o_ref):
    h = hin_ref[...].reshape(RB, H)
    a = (a0_ref[...].reshape(RB, H) + a1_ref[...].reshape(RB, H)) * (1.0 / EWS)
    z = sc_ref[...].reshape(1, H) * h + a
    t = jnp.maximum(jnp.dot(z, W1_ref[...].reshape(H, H),
                            preferred_element_type=jnp.float32)
                    + b1_ref[...].reshape(1, H), 0.0)
    t = (jnp.dot(t, W2_ref[...].reshape(H, H),
                 preferred_element_type=jnp.float32)
         + b2_ref[...].reshape(1, H))
    t = jnp.maximum(gm_ref[...].reshape(1, H) * t
                    + bt_ref[...].reshape(1, H), 0.0)
    o_ref[...] = (t + h).reshape(1, RB, H)


def _dense_layer(l, hin, aggp, W1, b1, W2, b2, gamma, beta, eps):
    RB = 2000
    NB = N // RB
    scl = jnp.broadcast_to((1.0 + eps[:, l])[:, None, None], (G, 1, H))

    def v3(p):
        return p[:, l][:, None, :]

    if l == 0:
        hin_spec = pl.BlockSpec((RB, H), lambda g, i: (i, 0))
    else:
        hin_spec = pl.BlockSpec((1, RB, H), lambda g, i: (g, i, 0))

    vspec = pl.BlockSpec((1, 1, H), lambda g, i: (g, 0, 0))
    return pl.pallas_call(
        functools.partial(_dense_body, RB),
        grid=(G, NB),
        in_specs=[
            hin_spec,
            pl.BlockSpec((1, 1, RB, H), lambda g, i: (0, g, i, 0)),
            pl.BlockSpec((1, 1, RB, H), lambda g, i: (1, g, i, 0)),
            vspec,
            pl.BlockSpec((1, H, H), lambda g, i: (g, 0, 0)),
            vspec,
            pl.BlockSpec((1, H, H), lambda g, i: (g, 0, 0)),
            vspec,
            vspec,
            vspec,
        ],
        out_specs=pl.BlockSpec((1, RB, H), lambda g, i: (g, i, 0)),
        out_shape=jax.ShapeDtypeStruct((G, N, H), jnp.float32),
    )(hin, aggp, aggp, scl, W1[:, l], v3(b1), W2[:, l], v3(b2),
      v3(gamma), v3(beta))


def _pool_body(RB, h_ref, b_ref, s_ref, c_ref):
    g = pl.program_id(0)
    i = pl.program_id(1)
    h = h_ref[...].reshape(RB, H)
    b = b_ref[...]
    oh = (b == lax.broadcasted_iota(jnp.int32, (RB, NG), 1)).astype(jnp.float32)
    sblk = jnp.dot(oh.T, h, preferred_element_type=jnp.float32)

    @pl.when(i == 0)
    def _():
        s_ref[...] = jnp.zeros_like(s_ref)

    s_ref[...] += sblk.reshape(1, NG, H)

    @pl.when((g == 0) & (i == 0))
    def _():
        c_ref[...] = jnp.zeros_like(c_ref)

    @pl.when(g == 0)
    def _():
        cs = jnp.sum(oh, axis=0)
        c_ref[...] += jnp.broadcast_to(cs[:, None], (NG, H))


def _pool(h2, batch2d):
    RB = 2000
    NB = N // RB
    return pl.pallas_call(
        functools.partial(_pool_body, RB),
        grid=(G, NB),
        in_specs=[
            pl.BlockSpec((1, RB, H), lambda g, i: (g, i, 0)),
            pl.BlockSpec((RB, 1), lambda g, i: (i, 0)),
        ],
        out_specs=[
            pl.BlockSpec((1, NG, H), lambda g, i: (g, 0, 0)),
            pl.BlockSpec((NG, H), lambda g, i: (0, 0)),
        ],
        out_shape=[
            jax.ShapeDtypeStruct((G, NG, H), jnp.float32),
            jax.ShapeDtypeStruct((NG, H), jnp.float32),
        ],
    )(h2, batch2d)


def _head_body(s_ref, c_ref, Wm_ref, bm_ref, Wf_ref, bf_ref, o_ref):
    c = jnp.maximum(c_ref[...], 1.0)
    acc = jnp.zeros((NG, H), jnp.float32)
    for g in range(G):
        hg = s_ref[g] / c
        hg = jnp.maximum(jnp.dot(hg, Wm_ref[g],
                                 preferred_element_type=jnp.float32)
                         + bm_ref[g], 0.0)
        acc = acc + hg
    acc = acc * (1.0 / G)
    o_ref[...] = (jnp.dot(acc, Wf_ref[...],
                          preferred_element_type=jnp.float32) + bf_ref[...])


def _head(s, c, Wm, bm, Wf, bf):
    return pl.pallas_call(
        _head_body,
        out_shape=jax.ShapeDtypeStruct((NG, NCLS), jnp.float32),
    )(s, c, Wm, bm[:, None, :], Wf, bf.reshape(1, NCLS))


def kernel(x, edge_attr, edge_weight, cand_edge_attr, cand_edge_weight,
           W1, b1, W2, b2, gamma, beta, eps, We, be, Wm, bm, Wf, bf,
           edge_index, cand_edge_index, batch):
    ei_all = jnp.concatenate([edge_index[None], cand_edge_index], axis=0)
    src_all = ei_all[:, 0, :].reshape(3 * E)
    dst_all = ei_all[:, 1, :].reshape(3 * E)
    ew_all = jnp.concatenate([edge_weight[None], cand_edge_weight],
                             axis=0).reshape(3 * E)
    ea_all = jnp.concatenate([edge_attr[None], cand_edge_attr],
                             axis=0).reshape(3 * E, DE)
    pk = jnp.stack([
        src_all.reshape(3 * QPB, CH),
        dst_all.reshape(3 * QPB, CH),
        (ew_all * EWS).astype(jnp.int32).reshape(3 * QPB, CH),
    ], axis=1)

    eenc = _edge_encode(ea_all, We, be)

    msg0 = _make_msg_call(N, (0, 0, 0))
    aggp0 = msg0(pk, eenc, x)
    h1 = _dense_layer(0, x, aggp0, W1, b1, W2, b2, gamma, beta, eps)

    msg1 = _make_msg_call(G * N, (0, N, 2 * N))
    aggp1 = msg1(pk, eenc, h1.reshape(G * N, H))
    h2 = _dense_layer(1, h1, aggp1, W1, b1, W2, b2, gamma, beta, eps)

    s, c = _pool(h2, batch.reshape(N, 1).astype(jnp.int32))
    return _head(s, c, Wm, bm, Wf, bf)


# speedup vs baseline: 1.3561x; 1.1747x over previous
import functools

import jax
import jax.numpy as jnp
from jax import lax
from jax.experimental import pallas as pl
from jax.experimental.pallas import tpu as pltpu
from jax.experimental.pallas import tpu_sc as plsc

N = 10000
E = 160000
H = 128
DE = 16
NC = 2
NL = 2
NG = 128
NCLS = 10
G = NC + 1

CH = 128
EPC = E // 2
QPC = EPC // CH
NSUB = 16
QPW = (QPC + NSUB - 1) // NSUB
RPS = 624
QPB = E // CH
EWS = 16777216.0


def _make_msg_call(table_rows, offs):
    mesh = plsc.VectorSubcoreMesh(core_axis_name="c", subcore_axis_name="s")

    @functools.partial(
        pl.kernel,
        mesh=mesh,
        out_type=jax.ShapeDtypeStruct((2, G, N, H), jnp.float32),
        scratch_types=[
            pltpu.VMEM((3, CH), jnp.int32),
            pltpu.VMEM((CH, H), jnp.float32),
            pltpu.VMEM((CH, H), jnp.float32),
            pltpu.VMEM_SHARED((N, H), jnp.float32),
            pltpu.SemaphoreType.DMA,
            pltpu.SemaphoreType.DMA,
        ],
    )
    def msg(pk_hbm, eenc_hbm, table_hbm, out_hbm,
            pk_v, rows_v, ea_v, agg_sh, s_g, s_ea):
        cid = lax.axis_index("c")
        sid = lax.axis_index("s")

        rbase = pl.multiple_of(sid * RPS, 8)

        for g in range(G):
            goff = jnp.int32(offs[g])

            z16 = jnp.zeros((16,), jnp.float32)

            def zrow(i, _):
                for k in range(H // 16):
                    rows_v[i, pl.ds(k * 16, 16)] = z16
                return ()

            lax.fori_loop(0, CH, zrow, ())
            zsrc = rows_v.at[pl.ds(0, CH)]
            for j in range(RPS // CH):
                pltpu.sync_copy(zsrc, agg_sh.at[pl.ds(rbase + j * CH, CH)])
            if RPS % CH:
                pltpu.sync_copy(rows_v.at[pl.ds(0, RPS % CH)],
                                agg_sh.at[pl.ds(rbase + (RPS // CH) * CH,
                                                RPS % CH)])

            @pl.when(sid == NSUB - 1)
            def _():
                pltpu.sync_copy(rows_v.at[pl.ds(0, 16)],
                                agg_sh.at[pl.ds(NSUB * RPS, 16)])

            plsc.subcore_barrier()

            def chunk(t, _):
                q = t * NSUB + sid

                @pl.when(q < QPC)
                def _():
                    Q = g * QPB + cid * QPC + q
                    pltpu.sync_copy(pk_hbm.at[Q], pk_v)
                    if offs[g]:
                        for k in range(CH // 16):
                            sl = pl.ds(k * 16, 16)
                            pk_v[0, sl] = pk_v[0, sl] + goff
                    gat = pltpu.async_copy(table_hbm.at[pk_v.at[0]],
                                           rows_v, s_g)
                    eac = pltpu.async_copy(
                        eenc_hbm.at[pl.ds(pl.multiple_of(Q * CH, CH), CH)],
                        ea_v, s_ea)
                    gat.wait()
                    eac.wait()

                    def egroup(gi, _):
                        e0 = gi * 16
                        wv = pk_v[2, pl.ds(e0, 16)].astype(jnp.float32)
                        for j in range(16):
                            wj = jnp.full((16,), wv[j], jnp.float32)
                            e = e0 + j
                            for k in range(H // 16):
                                sl = pl.ds(k * 16, 16)
                                rows_v[e, sl] = jnp.maximum(
                                    rows_v[e, sl] + ea_v[e, sl], 0.0) * wj
                        return ()

                    lax.fori_loop(0, CH // 16, egroup, ())
                    pltpu.sync_copy(rows_v, agg_sh.at[pk_v.at[1]], add=True)

                return ()

            lax.fori_loop(0, QPW, chunk, ())
            plsc.subcore_barrier()
            pltpu.sync_copy(agg_sh.at[pl.ds(rbase, RPS)],
                            out_hbm.at[cid, g, pl.ds(rbase, RPS)])

            @pl.when(sid == NSUB - 1)
            def _():
                pltpu.sync_copy(agg_sh.at[pl.ds(NSUB * RPS, 16)],
                                out_hbm.at[cid, g, pl.ds(NSUB * RPS, 16)])

            plsc.subcore_barrier()

    return msg


def _eenc_body(ea_ref, We_ref, be_ref, o_ref):
    o_ref[...] = (jnp.dot(ea_ref[...], We_ref[...],
                          preferred_element_type=jnp.float32) + be_ref[...])


def _edge_encode(ea_all, We, be):
    RB = 4000
    grid = (3 * E // RB,)
    return pl.pallas_call(
        _eenc_body,
        grid=grid,
        in_specs=[
            pl.BlockSpec((RB, DE), lambda i: (i, 0)),
            pl.BlockSpec((DE, H), lambda i: (0, 0)),
            pl.BlockSpec((1, H), lambda i: (0, 0)),
        ],
        out_specs=pl.BlockSpec((RB, H), lambda i: (i, 0)),
        out_shape=jax.ShapeDtypeStruct((3 * E, H), jnp.float32),
    )(ea_all, We, be.reshape(1, H))


def _dense_body(RB, hin_ref, a0_ref, a1_ref, sc_ref, W1_ref, b1_ref,
                W2_ref, b2_ref, gm_ref, bt_ref, o_ref):
    h = hin_ref[...].reshape(RB, H)
    a = (a0_ref[...].reshape(RB, H) + a1_ref[...].reshape(RB, H)) * (1.0 / EWS)
    z = sc_ref[...].reshape(1, H) * h + a
    t = jnp.maximum(jnp.dot(z, W1_ref[...].reshape(H, H),
                            preferred_element_type=jnp.float32)
                    + b1_ref[...].reshape(1, H), 0.0)
    t = (jnp.dot(t, W2_ref[...].reshape(H, H),
                 preferred_element_type=jnp.float32)
         + b2_ref[...].reshape(1, H))
    t = jnp.maximum(gm_ref[...].reshape(1, H) * t
                    + bt_ref[...].reshape(1, H), 0.0)
    o_ref[...] = (t + h).reshape(1, RB, H)


def _dense_layer(l, hin, aggp, W1, b1, W2, b2, gamma, beta, eps):
    RB = 2000
    NB = N // RB
    scl = jnp.broadcast_to((1.0 + eps[:, l])[:, None, None], (G, 1, H))

    def v3(p):
        return p[:, l][:, None, :]

    if l == 0:
        hin_spec = pl.BlockSpec((RB, H), lambda g, i: (i, 0))
    else:
        hin_spec = pl.BlockSpec((1, RB, H), lambda g, i: (g, i, 0))

    vspec = pl.BlockSpec((1, 1, H), lambda g, i: (g, 0, 0))
    return pl.pallas_call(
        functools.partial(_dense_body, RB),
        grid=(G, NB),
        in_specs=[
            hin_spec,
            pl.BlockSpec((1, 1, RB, H), lambda g, i: (0, g, i, 0)),
            pl.BlockSpec((1, 1, RB, H), lambda g, i: (1, g, i, 0)),
            vspec,
            pl.BlockSpec((1, H, H), lambda g, i: (g, 0, 0)),
            vspec,
            pl.BlockSpec((1, H, H), lambda g, i: (g, 0, 0)),
            vspec,
            vspec,
            vspec,
        ],
        out_specs=pl.BlockSpec((1, RB, H), lambda g, i: (g, i, 0)),
        out_shape=jax.ShapeDtypeStruct((G, N, H), jnp.float32),
    )(hin, aggp, aggp, scl, W1[:, l], v3(b1), W2[:, l], v3(b2),
      v3(gamma), v3(beta))


def _pool_body(RB, h_ref, b_ref, s_ref, c_ref):
    g = pl.program_id(0)
    i = pl.program_id(1)
    h = h_ref[...].reshape(RB, H)
    b = b_ref[...]
    oh = (b == lax.broadcasted_iota(jnp.int32, (RB, NG), 1)).astype(jnp.float32)
    sblk = jnp.dot(oh.T, h, preferred_element_type=jnp.float32)

    @pl.when(i == 0)
    def _():
        s_ref[...] = jnp.zeros_like(s_ref)

    s_ref[...] += sblk.reshape(1, NG, H)

    @pl.when((g == 0) & (i == 0))
    def _():
        c_ref[...] = jnp.zeros_like(c_ref)

    @pl.when(g == 0)
    def _():
        cs = jnp.sum(oh, axis=0)
        c_ref[...] += jnp.broadcast_to(cs[:, None], (NG, H))


def _pool(h2, batch2d):
    RB = 2000
    NB = N // RB
    return pl.pallas_call(
        functools.partial(_pool_body, RB),
        grid=(G, NB),
        in_specs=[
            pl.BlockSpec((1, RB, H), lambda g, i: (g, i, 0)),
            pl.BlockSpec((RB, 1), lambda g, i: (i, 0)),
        ],
        out_specs=[
            pl.BlockSpec((1, NG, H), lambda g, i: (g, 0, 0)),
            pl.BlockSpec((NG, H), lambda g, i: (0, 0)),
        ],
        out_shape=[
            jax.ShapeDtypeStruct((G, NG, H), jnp.float32),
            jax.ShapeDtypeStruct((NG, H), jnp.float32),
        ],
    )(h2, batch2d)


def _head_body(s_ref, c_ref, Wm_ref, bm_ref, Wf_ref, bf_ref, o_ref):
    c = jnp.maximum(c_ref[...], 1.0)
    acc = jnp.zeros((NG, H), jnp.float32)
    for g in range(G):
        hg = s_ref[g] / c
        hg = jnp.maximum(jnp.dot(hg, Wm_ref[g],
                                 preferred_element_type=jnp.float32)
                         + bm_ref[g], 0.0)
        acc = acc + hg
    acc = acc * (1.0 / G)
    o_ref[...] = (jnp.dot(acc, Wf_ref[...],
                          preferred_element_type=jnp.float32) + bf_ref[...])


def _head(s, c, Wm, bm, Wf, bf):
    return pl.pallas_call(
        _head_body,
        out_shape=jax.ShapeDtypeStruct((NG, NCLS), jnp.float32),
    )(s, c, Wm, bm[:, None, :], Wf, bf.reshape(1, NCLS))


def kernel(x, edge_attr, edge_weight, cand_edge_attr, cand_edge_weight,
           W1, b1, W2, b2, gamma, beta, eps, We, be, Wm, bm, Wf, bf,
           edge_index, cand_edge_index, batch):
    ei_all = jnp.concatenate([edge_index[None], cand_edge_index], axis=0)
    src_all = ei_all[:, 0, :].reshape(3 * E)
    dst_all = ei_all[:, 1, :].reshape(3 * E)
    ew_all = jnp.concatenate([edge_weight[None], cand_edge_weight],
                             axis=0).reshape(3 * E)
    ea_all = jnp.concatenate([edge_attr[None], cand_edge_attr],
                             axis=0).reshape(3 * E, DE)
    pk = jnp.stack([
        src_all.reshape(3 * QPB, CH),
        dst_all.reshape(3 * QPB, CH),
        (ew_all * EWS).astype(jnp.int32).reshape(3 * QPB, CH),
    ], axis=1)

    eenc = _edge_encode(ea_all, We, be)

    msg0 = _make_msg_call(N, (0, 0, 0))
    aggp0 = msg0(pk, eenc, x)
    h1 = _dense_layer(0, x, aggp0, W1, b1, W2, b2, gamma, beta, eps)

    msg1 = _make_msg_call(G * N, (0, N, 2 * N))
    aggp1 = msg1(pk, eenc, h1.reshape(G * N, H))
    h2 = _dense_layer(1, h1, aggp1, W1, b1, W2, b2, gamma, beta, eps)

    s, c = _pool(h2, batch.reshape(N, 1).astype(jnp.int32))
    return _head(s, c, Wm, bm, Wf, bf)


# speedup vs baseline: 1.4106x; 1.0402x over previous
import functools

import jax
import jax.numpy as jnp
from jax import lax
from jax.experimental import pallas as pl
from jax.experimental.pallas import tpu as pltpu
from jax.experimental.pallas import tpu_sc as plsc

N = 10000
E = 160000
H = 128
DE = 16
NC = 2
NL = 2
NG = 128
NCLS = 10
G = NC + 1

CH = 128
EPC = E // 2
QPC = EPC // CH
NSUB = 16
QPW = (QPC + NSUB - 1) // NSUB
RPS = 624
QPB = E // CH
EWS = 16777216.0
PKF = 8
QPCP = QPW * NSUB


def _make_msg_call(table_rows, offs):
    mesh = plsc.VectorSubcoreMesh(core_axis_name="c", subcore_axis_name="s")

    @functools.partial(
        pl.kernel,
        mesh=mesh,
        out_type=jax.ShapeDtypeStruct((2, G, N, H), jnp.float32),
        scratch_types=[
            pltpu.VMEM((PKF * 3, CH), jnp.int32),
            pltpu.VMEM((CH, H), jnp.float32),
            pltpu.VMEM((CH, H), jnp.float32),
            pltpu.VMEM_SHARED((N, H), jnp.float32),
            pltpu.SemaphoreType.DMA,
            pltpu.SemaphoreType.DMA,
        ],
    )
    def msg(pk_hbm, eenc_hbm, table_hbm, out_hbm,
            pk_v, rows_v, ea_v, agg_sh, s_g, s_ea):
        cid = lax.axis_index("c")
        sid = lax.axis_index("s")

        rbase = pl.multiple_of(sid * RPS, 8)

        for g in range(G):
            goff = jnp.int32(offs[g])

            z16 = jnp.zeros((16,), jnp.float32)

            def zrow(i, _):
                for k in range(H // 16):
                    rows_v[i, pl.ds(k * 16, 16)] = z16
                return ()

            lax.fori_loop(0, CH, zrow, ())
            zsrc = rows_v.at[pl.ds(0, CH)]
            for j in range(RPS // CH):
                pltpu.sync_copy(zsrc, agg_sh.at[pl.ds(rbase + j * CH, CH)])
            if RPS % CH:
                pltpu.sync_copy(rows_v.at[pl.ds(0, RPS % CH)],
                                agg_sh.at[pl.ds(rbase + (RPS // CH) * CH,
                                                RPS % CH)])

            @pl.when(sid == NSUB - 1)
            def _():
                pltpu.sync_copy(rows_v.at[pl.ds(0, 16)],
                                agg_sh.at[pl.ds(NSUB * RPS, 16)])

            plsc.subcore_barrier()

            wbase = pl.multiple_of(
                (((g * 2 + cid) * NSUB + sid) * QPW) * 3, 24)

            def chunk(t, _):
                q = t * NSUB + sid
                p = jnp.int32(t) % PKF

                @pl.when(p == 0)
                def _():
                    pltpu.sync_copy(
                        pk_hbm.at[pl.ds(
                            pl.multiple_of(wbase + t * 3, 24), PKF * 3)],
                        pk_v)

                @pl.when(q < QPC)
                def _():
                    Q = g * QPB + cid * QPC + q
                    p3 = p * 3
                    if offs[g]:
                        for k in range(CH // 16):
                            sl = pl.ds(k * 16, 16)
                            pk_v[p3, sl] = pk_v[p3, sl] + goff
                    gat = pltpu.async_copy(table_hbm.at[pk_v.at[p3]],
                                           rows_v, s_g)
                    eac = pltpu.async_copy(
                        eenc_hbm.at[pl.ds(pl.multiple_of(Q * CH, CH), CH)],
                        ea_v, s_ea)
                    gat.wait()
                    eac.wait()

                    def egroup(gi, _):
                        e0 = gi * 16
                        wv = pk_v[p3 + 2, pl.ds(e0, 16)].astype(jnp.float32)
                        for j in range(16):
                            wj = jnp.full((16,), wv[j], jnp.float32)
                            e = e0 + j
                            for k in range(H // 16):
                                sl = pl.ds(k * 16, 16)
                                rows_v[e, sl] = jnp.maximum(
                                    rows_v[e, sl] + ea_v[e, sl], 0.0) * wj
                        return ()

                    lax.fori_loop(0, CH // 16, egroup, ())
                    pltpu.sync_copy(rows_v, agg_sh.at[pk_v.at[p3 + 1]],
                                    add=True)

                return ()

            lax.fori_loop(0, QPW, chunk, ())
            plsc.subcore_barrier()
            pltpu.sync_copy(agg_sh.at[pl.ds(rbase, RPS)],
                            out_hbm.at[cid, g, pl.ds(rbase, RPS)])

            @pl.when(sid == NSUB - 1)
            def _():
                pltpu.sync_copy(agg_sh.at[pl.ds(NSUB * RPS, 16)],
                                out_hbm.at[cid, g, pl.ds(NSUB * RPS, 16)])

            plsc.subcore_barrier()

    return msg


def _eenc_body(ea_ref, We_ref, be_ref, o_ref):
    o_ref[...] = (jnp.dot(ea_ref[...], We_ref[...],
                          preferred_element_type=jnp.float32) + be_ref[...])


def _edge_encode(ea_all, We, be):
    RB = 4000
    grid = (3 * E // RB,)
    return pl.pallas_call(
        _eenc_body,
        grid=grid,
        in_specs=[
            pl.BlockSpec((RB, DE), lambda i: (i, 0)),
            pl.BlockSpec((DE, H), lambda i: (0, 0)),
            pl.BlockSpec((1, H), lambda i: (0, 0)),
        ],
        out_specs=pl.BlockSpec((RB, H), lambda i: (i, 0)),
        out_shape=jax.ShapeDtypeStruct((3 * E, H), jnp.float32),
    )(ea_all, We, be.reshape(1, H))


def _dense_body(RB, hin_ref, a0_ref, a1_ref, sc_ref, W1_ref, b1_ref,
                W2_ref, b2_ref, gm_ref, bt_ref, o_ref):
    h = hin_ref[...].reshape(RB, H)
    a = (a0_ref[...].reshape(RB, H) + a1_ref[...].reshape(RB, H)) * (1.0 / EWS)
    z = sc_ref[...].reshape(1, H) * h + a
    t = jnp.maximum(jnp.dot(z, W1_ref[...].reshape(H, H),
                            preferred_element_type=jnp.float32)
                    + b1_ref[...].reshape(1, H), 0.0)
    t = (jnp.dot(t, W2_ref[...].reshape(H, H),
                 preferred_element_type=jnp.float32)
         + b2_ref[...].reshape(1, H))
    t = jnp.maximum(gm_ref[...].reshape(1, H) * t
                    + bt_ref[...].reshape(1, H), 0.0)
    o_ref[...] = (t + h).reshape(1, RB, H)


def _dense_layer(l, hin, aggp, W1, b1, W2, b2, gamma, beta, eps):
    RB = 2000
    NB = N // RB
    scl = jnp.broadcast_to((1.0 + eps[:, l])[:, None, None], (G, 1, H))

    def v3(p):
        return p[:, l][:, None, :]

    if l == 0:
        hin_spec = pl.BlockSpec((RB, H), lambda g, i: (i, 0))
    else:
        hin_spec = pl.BlockSpec((1, RB, H), lambda g, i: (g, i, 0))

    vspec = pl.BlockSpec((1, 1, H), lambda g, i: (g, 0, 0))
    return pl.pallas_call(
        functools.partial(_dense_body, RB),
        grid=(G, NB),
        in_specs=[
            hin_spec,
            pl.BlockSpec((1, 1, RB, H), lambda g, i: (0, g, i, 0)),
            pl.BlockSpec((1, 1, RB, H), lambda g, i: (1, g, i, 0)),
            vspec,
            pl.BlockSpec((1, H, H), lambda g, i: (g, 0, 0)),
            vspec,
            pl.BlockSpec((1, H, H), lambda g, i: (g, 0, 0)),
            vspec,
            vspec,
            vspec,
        ],
        out_specs=pl.BlockSpec((1, RB, H), lambda g, i: (g, i, 0)),
        out_shape=jax.ShapeDtypeStruct((G, N, H), jnp.float32),
    )(hin, aggp, aggp, scl, W1[:, l], v3(b1), W2[:, l], v3(b2),
      v3(gamma), v3(beta))


def _pool_body(RB, h_ref, b_ref, s_ref, c_ref):
    g = pl.program_id(0)
    i = pl.program_id(1)
    h = h_ref[...].reshape(RB, H)
    b = b_ref[...]
    oh = (b == lax.broadcasted_iota(jnp.int32, (RB, NG), 1)).astype(jnp.float32)
    sblk = jnp.dot(oh.T, h, preferred_element_type=jnp.float32)

    @pl.when(i == 0)
    def _():
        s_ref[...] = jnp.zeros_like(s_ref)

    s_ref[...] += sblk.reshape(1, NG, H)

    @pl.when((g == 0) & (i == 0))
    def _():
        c_ref[...] = jnp.zeros_like(c_ref)

    @pl.when(g == 0)
    def _():
        cs = jnp.sum(oh, axis=0)
        c_ref[...] += jnp.broadcast_to(cs[:, None], (NG, H))


def _pool(h2, batch2d):
    RB = 2000
    NB = N // RB
    return pl.pallas_call(
        functools.partial(_pool_body, RB),
        grid=(G, NB),
        in_specs=[
            pl.BlockSpec((1, RB, H), lambda g, i: (g, i, 0)),
            pl.BlockSpec((RB, 1), lambda g, i: (i, 0)),
        ],
        out_specs=[
            pl.BlockSpec((1, NG, H), lambda g, i: (g, 0, 0)),
            pl.BlockSpec((NG, H), lambda g, i: (0, 0)),
        ],
        out_shape=[
            jax.ShapeDtypeStruct((G, NG, H), jnp.float32),
            jax.ShapeDtypeStruct((NG, H), jnp.float32),
        ],
    )(h2, batch2d)


def _head_body(s_ref, c_ref, Wm_ref, bm_ref, Wf_ref, bf_ref, o_ref):
    c = jnp.maximum(c_ref[...], 1.0)
    acc = jnp.zeros((NG, H), jnp.float32)
    for g in range(G):
        hg = s_ref[g] / c
        hg = jnp.maximum(jnp.dot(hg, Wm_ref[g],
                                 preferred_element_type=jnp.float32)
                         + bm_ref[g], 0.0)
        acc = acc + hg
    acc = acc * (1.0 / G)
    o_ref[...] = (jnp.dot(acc, Wf_ref[...],
                          preferred_element_type=jnp.float32) + bf_ref[...])


def _head(s, c, Wm, bm, Wf, bf):
    return pl.pallas_call(
        _head_body,
        out_shape=jax.ShapeDtypeStruct((NG, NCLS), jnp.float32),
    )(s, c, Wm, bm[:, None, :], Wf, bf.reshape(1, NCLS))


def kernel(x, edge_attr, edge_weight, cand_edge_attr, cand_edge_weight,
           W1, b1, W2, b2, gamma, beta, eps, We, be, Wm, bm, Wf, bf,
           edge_index, cand_edge_index, batch):
    ei_all = jnp.concatenate([edge_index[None], cand_edge_index], axis=0)
    src_all = ei_all[:, 0, :].reshape(3 * E)
    dst_all = ei_all[:, 1, :].reshape(3 * E)
    ew_all = jnp.concatenate([edge_weight[None], cand_edge_weight],
                             axis=0).reshape(3 * E)
    ea_all = jnp.concatenate([edge_attr[None], cand_edge_attr],
                             axis=0).reshape(3 * E, DE)
    pk = jnp.stack([
        src_all.reshape(3 * QPB, CH),
        dst_all.reshape(3 * QPB, CH),
        (ew_all * EWS).astype(jnp.int32).reshape(3 * QPB, CH),
    ], axis=1)
    pkp = jnp.pad(pk.reshape(G, 2, QPC, 3, CH),
                  ((0, 0), (0, 0), (0, QPCP - QPC), (0, 0), (0, 0)))
    pk_w = pkp.reshape(G, 2, QPW, NSUB, 3, CH).transpose(0, 1, 3, 2, 4, 5)
    pk_w = pk_w.reshape(G * 2 * NSUB * QPW * 3, CH)

    eenc = _edge_encode(ea_all, We, be)

    msg0 = _make_msg_call(N, (0, 0, 0))
    aggp0 = msg0(pk_w, eenc, x)
    h1 = _dense_layer(0, x, aggp0, W1, b1, W2, b2, gamma, beta, eps)

    msg1 = _make_msg_call(G * N, (0, N, 2 * N))
    aggp1 = msg1(pk_w, eenc, h1.reshape(G * N, H))
    h2 = _dense_layer(1, h1, aggp1, W1, b1, W2, b2, gamma, beta, eps)

    s, c = _pool(h2, batch.reshape(N, 1).astype(jnp.int32))
    return _head(s, c, Wm, bm, Wf, bf)


# speedup vs baseline: 1.4111x; 1.0003x over previous
import functools

import jax
import jax.numpy as jnp
from jax import lax
from jax.experimental import pallas as pl
from jax.experimental.pallas import tpu as pltpu
from jax.experimental.pallas import tpu_sc as plsc

N = 10000
E = 160000
H = 128
DE = 16
NC = 2
NL = 2
NG = 128
NCLS = 10
G = NC + 1

CH = 128
EPC = E // 2
QPC = EPC // CH
NSUB = 16
QPW = (QPC + NSUB - 1) // NSUB
RPS = 624
QPB = E // CH
EWS = 16777216.0
PKF = 8
QPCP = QPW * NSUB


def _make_msg_call(table_rows, offs):
    mesh = plsc.VectorSubcoreMesh(core_axis_name="c", subcore_axis_name="s")

    @functools.partial(
        pl.kernel,
        mesh=mesh,
        out_type=jax.ShapeDtypeStruct((2, G, N, H), jnp.float32),
        scratch_types=[
            pltpu.VMEM((PKF * 3, CH), jnp.int32),
            pltpu.VMEM((CH, H), jnp.float32),
            pltpu.VMEM((CH, H), jnp.float32),
            pltpu.VMEM_SHARED((N, H), jnp.float32),
            pltpu.SemaphoreType.DMA,
            pltpu.SemaphoreType.DMA,
        ],
    )
    def msg(pk_hbm, eenc_hbm, table_hbm, out_hbm,
            pk_v, rows_v, ea_v, agg_sh, s_g, s_ea):
        cid = lax.axis_index("c")
        sid = lax.axis_index("s")

        rbase = pl.multiple_of(sid * RPS, 8)

        for g in range(G):
            goff = jnp.int32(offs[g])

            z16 = jnp.zeros((16,), jnp.float32)

            def zrow(i, _):
                for k in range(H // 16):
                    rows_v[i, pl.ds(k * 16, 16)] = z16
                return ()

            lax.fori_loop(0, CH, zrow, ())
            zsrc = rows_v.at[pl.ds(0, CH)]
            for j in range(RPS // CH):
                pltpu.sync_copy(zsrc, agg_sh.at[pl.ds(rbase + j * CH, CH)])
            if RPS % CH:
                pltpu.sync_copy(rows_v.at[pl.ds(0, RPS % CH)],
                                agg_sh.at[pl.ds(rbase + (RPS // CH) * CH,
                                                RPS % CH)])

            @pl.when(sid == NSUB - 1)
            def _():
                pltpu.sync_copy(rows_v.at[pl.ds(0, 16)],
                                agg_sh.at[pl.ds(NSUB * RPS, 16)])

            plsc.subcore_barrier()

            wbase = pl.multiple_of(
                (((g * 2 + cid) * NSUB + sid) * QPW) * 3, 24)

            def chunk(t, _):
                q = t * NSUB + sid
                p = jnp.int32(t) % PKF

                @pl.when(p == 0)
                def _():
                    pltpu.sync_copy(
                        pk_hbm.at[pl.ds(
                            pl.multiple_of(wbase + t * 3, 24), PKF * 3)],
                        pk_v)

                @pl.when(q < QPC)
                def _():
                    Q = g * QPB + cid * QPC + q
                    p3 = p * 3
                    if offs[g]:
                        for k in range(CH // 16):
                            sl = pl.ds(k * 16, 16)
                            pk_v[p3, sl] = pk_v[p3, sl] + goff
                    gat = pltpu.async_copy(table_hbm.at[pk_v.at[p3]],
                                           rows_v, s_g)
                    eac = pltpu.async_copy(
                        eenc_hbm.at[pl.ds(pl.multiple_of(Q * CH, CH), CH)],
                        ea_v, s_ea)
                    gat.wait()
                    eac.wait()

                    def egroup(gi, _):
                        e0 = gi * 16
                        wv = pk_v[p3 + 2, pl.ds(e0, 16)].astype(jnp.float32)
                        for j in range(16):
                            wj = jnp.full((16,), wv[j], jnp.float32)
                            e = e0 + j
                            for k in range(H // 16):
                                sl = pl.ds(k * 16, 16)
                                rows_v[e, sl] = jnp.maximum(
                                    rows_v[e, sl] + ea_v[e, sl], 0.0) * wj
                        return ()

                    lax.fori_loop(0, CH // 16, egroup, ())
                    pltpu.sync_copy(rows_v, agg_sh.at[pk_v.at[p3 + 1]],
                                    add=True)

                return ()

            lax.fori_loop(0, QPW, chunk, ())
            plsc.subcore_barrier()
            pltpu.sync_copy(agg_sh.at[pl.ds(rbase, RPS)],
                            out_hbm.at[cid, g, pl.ds(rbase, RPS)])

            @pl.when(sid == NSUB - 1)
            def _():
                pltpu.sync_copy(agg_sh.at[pl.ds(NSUB * RPS, 16)],
                                out_hbm.at[cid, g, pl.ds(NSUB * RPS, 16)])

            plsc.subcore_barrier()

    return msg


def _eenc_body(ea_ref, We_ref, be_ref, o_ref):
    o_ref[...] = (jnp.dot(ea_ref[...], We_ref[...],
                          preferred_element_type=jnp.float32) + be_ref[...])


def _edge_encode(ea_all, We, be):
    RB = 4000
    grid = (3 * E // RB,)
    return pl.pallas_call(
        _eenc_body,
        grid=grid,
        in_specs=[
            pl.BlockSpec((RB, DE), lambda i: (i, 0)),
            pl.BlockSpec((DE, H), lambda i: (0, 0)),
            pl.BlockSpec((1, H), lambda i: (0, 0)),
        ],
        out_specs=pl.BlockSpec((RB, H), lambda i: (i, 0)),
        out_shape=jax.ShapeDtypeStruct((3 * E, H), jnp.float32),
    )(ea_all, We, be.reshape(1, H))


def _dense_body(RB, hin_ref, a0_ref, a1_ref, sc_ref, W1_ref, b1_ref,
                W2_ref, b2_ref, gm_ref, bt_ref, o_ref):
    h = hin_ref[...].reshape(RB, H)
    a = (a0_ref[...].reshape(RB, H) + a1_ref[...].reshape(RB, H)) * (1.0 / EWS)
    z = sc_ref[...].reshape(1, H) * h + a
    t = jnp.maximum(jnp.dot(z, W1_ref[...].reshape(H, H),
                            preferred_element_type=jnp.float32)
                    + b1_ref[...].reshape(1, H), 0.0)
    t = (jnp.dot(t, W2_ref[...].reshape(H, H),
                 preferred_element_type=jnp.float32)
         + b2_ref[...].reshape(1, H))
    t = jnp.maximum(gm_ref[...].reshape(1, H) * t
                    + bt_ref[...].reshape(1, H), 0.0)
    o_ref[...] = t + h


def _dense_layer(l, hin, aggp, W1, b1, W2, b2, gamma, beta, eps):
    RB = 2000
    NB = N // RB
    scl = jnp.broadcast_to((1.0 + eps[:, l])[:, None, None], (G, 1, H))

    def v3(p):
        return p[:, l][:, None, :]

    if l == 0:
        hin_spec = pl.BlockSpec((RB, H), lambda g, i: (i, 0))
    else:
        hin_spec = pl.BlockSpec((RB, H), lambda g, i: (g * (N // 2000) + i, 0))

    vspec = pl.BlockSpec((1, 1, H), lambda g, i: (g, 0, 0))
    return pl.pallas_call(
        functools.partial(_dense_body, RB),
        grid=(G, NB),
        in_specs=[
            hin_spec,
            pl.BlockSpec((1, 1, RB, H), lambda g, i: (0, g, i, 0)),
            pl.BlockSpec((1, 1, RB, H), lambda g, i: (1, g, i, 0)),
            vspec,
            pl.BlockSpec((1, H, H), lambda g, i: (g, 0, 0)),
            vspec,
            pl.BlockSpec((1, H, H), lambda g, i: (g, 0, 0)),
            vspec,
            vspec,
            vspec,
        ],
        out_specs=pl.BlockSpec((RB, H), lambda g, i: (g * NB + i, 0)),
        out_shape=jax.ShapeDtypeStruct((G * N, H), jnp.float32),
    )(hin, aggp, aggp, scl, W1[:, l], v3(b1), W2[:, l], v3(b2),
      v3(gamma), v3(beta))


def _pool_body(RB, h_ref, b_ref, s_ref, c_ref):
    g = pl.program_id(0)
    i = pl.program_id(1)
    h = h_ref[...]
    b = b_ref[...]
    oh = (b == lax.broadcasted_iota(jnp.int32, (RB, NG), 1)).astype(jnp.float32)
    sblk = jnp.dot(oh.T, h, preferred_element_type=jnp.float32)

    @pl.when(i == 0)
    def _():
        s_ref[...] = jnp.zeros_like(s_ref)

    s_ref[...] += sblk.reshape(1, NG, H)

    @pl.when((g == 0) & (i == 0))
    def _():
        c_ref[...] = jnp.zeros_like(c_ref)

    @pl.when(g == 0)
    def _():
        cs = jnp.sum(oh, axis=0)
        c_ref[...] += jnp.broadcast_to(cs[:, None], (NG, H))


def _pool(h2, batch2d):
    RB = 2000
    NB = N // RB
    return pl.pallas_call(
        functools.partial(_pool_body, RB),
        grid=(G, NB),
        in_specs=[
            pl.BlockSpec((RB, H), lambda g, i: (g * (N // 2000) + i, 0)),
            pl.BlockSpec((RB, 1), lambda g, i: (i, 0)),
        ],
        out_specs=[
            pl.BlockSpec((1, NG, H), lambda g, i: (g, 0, 0)),
            pl.BlockSpec((NG, H), lambda g, i: (0, 0)),
        ],
        out_shape=[
            jax.ShapeDtypeStruct((G, NG, H), jnp.float32),
            jax.ShapeDtypeStruct((NG, H), jnp.float32),
        ],
    )(h2, batch2d)


def _head_body(s_ref, c_ref, Wm_ref, bm_ref, Wf_ref, bf_ref, o_ref):
    c = jnp.maximum(c_ref[...], 1.0)
    acc = jnp.zeros((NG, H), jnp.float32)
    for g in range(G):
        hg = s_ref[g] / c
        hg = jnp.maximum(jnp.dot(hg, Wm_ref[g],
                                 preferred_element_type=jnp.float32)
                         + bm_ref[g], 0.0)
        acc = acc + hg
    acc = acc * (1.0 / G)
    o_ref[...] = (jnp.dot(acc, Wf_ref[...],
                          preferred_element_type=jnp.float32) + bf_ref[...])


def _head(s, c, Wm, bm, Wf, bf):
    return pl.pallas_call(
        _head_body,
        out_shape=jax.ShapeDtypeStruct((NG, NCLS), jnp.float32),
    )(s, c, Wm, bm[:, None, :], Wf, bf.reshape(1, NCLS))


def kernel(x, edge_attr, edge_weight, cand_edge_attr, cand_edge_weight,
           W1, b1, W2, b2, gamma, beta, eps, We, be, Wm, bm, Wf, bf,
           edge_index, cand_edge_index, batch):
    ei_all = jnp.concatenate([edge_index[None], cand_edge_index], axis=0)
    src_all = ei_all[:, 0, :].reshape(3 * E)
    dst_all = ei_all[:, 1, :].reshape(3 * E)
    ew_all = jnp.concatenate([edge_weight[None], cand_edge_weight],
                             axis=0).reshape(3 * E)
    ea_all = jnp.concatenate([edge_attr[None], cand_edge_attr],
                             axis=0).reshape(3 * E, DE)
    pk = jnp.stack([
        src_all.reshape(3 * QPB, CH),
        dst_all.reshape(3 * QPB, CH),
        (ew_all * EWS).astype(jnp.int32).reshape(3 * QPB, CH),
    ], axis=1)
    pkp = jnp.pad(pk.reshape(G, 2, QPC, 3, CH),
                  ((0, 0), (0, 0), (0, QPCP - QPC), (0, 0), (0, 0)))
    pk_w = pkp.reshape(G, 2, QPW, NSUB, 3, CH).transpose(0, 1, 3, 2, 4, 5)
    pk_w = pk_w.reshape(G * 2 * NSUB * QPW * 3, CH)

    eenc = _edge_encode(ea_all, We, be)

    msg0 = _make_msg_call(N, (0, 0, 0))
    aggp0 = msg0(pk_w, eenc, x)
    h1 = _dense_layer(0, x, aggp0, W1, b1, W2, b2, gamma, beta, eps)

    msg1 = _make_msg_call(G * N, (0, N, 2 * N))
    aggp1 = msg1(pk_w, eenc, h1)
    h2 = _dense_layer(1, h1, aggp1, W1, b1, W2, b2, gamma, beta, eps)

    s, c = _pool(h2, batch.reshape(N, 1).astype(jnp.int32))
    return _head(s, c, Wm, bm, Wf, bf)
